# baseline (speedup 1.0000x reference)
import functools

import jax
import jax.numpy as jnp
from jax import lax
from jax.experimental import pallas as pl
from jax.experimental.pallas import tpu as pltpu
from jax.experimental.pallas import tpu_sc as plsc

N = 10000
C = 128
E = 320000

NC = 2
NS = 16
NW = NC * NS
EW = E // NW
K = 64
EWP = 10304
E_PAD = EWP * NW
NCHUNK = EWP // K
RPT = N // NS
L = 16

_mesh = functools.partial(
    plsc.VectorSubcoreMesh, core_axis_name="c", subcore_axis_name="s")
_SC_PARAMS = pltpu.CompilerParams(needs_layout_passes=False,
                                  use_tc_tiling_on_sc=False)


def _deg_body(col_hbm, ew_hbm, out_hbm, col_v, ew_v, deg_v):
    c = lax.axis_index("c")
    s = lax.axis_index("s")
    wid = c * NS + s
    base = wid * EW
    pltpu.sync_copy(col_hbm.at[pl.ds(base, EW)], col_v)
    pltpu.sync_copy(ew_hbm.at[pl.ds(base, EW)], ew_v)

    def zero_body(i, carry):
        deg_v[pl.ds(i * L, L)] = jnp.zeros((L,), jnp.float32)
        return carry

    lax.fori_loop(0, N // L, zero_body, 0)

    def acc_body(i, carry):
        idx = col_v[pl.ds(i * L, L)]
        w = ew_v[pl.ds(i * L, L)]
        plsc.addupdate_scatter(deg_v, [idx], w)
        return carry

    lax.fori_loop(0, EW // L, acc_body, 0)
    pltpu.sync_copy(deg_v, out_hbm.at[wid])


@jax.jit
def _deg_call(col, ew):
    return pl.kernel(
        _deg_body,
        out_type=jax.ShapeDtypeStruct((NW, N), jnp.float32),
        mesh=_mesh(),
        compiler_params=_SC_PARAMS,
        scratch_types=[
            pltpu.VMEM((EW,), jnp.int32),
            pltpu.VMEM((EW,), jnp.float32),
            pltpu.VMEM((N,), jnp.float32),
        ],
    )(col, ew)


def _dense1_body(x_ref, iw_ref, wih_ref, whh_ref, bih_ref, bhh_ref, dp_ref,
                 h2_ref, dinv_ref):
    iw = iw_ref[...]
    gi = lax.dot_general(iw, wih_ref[...], (((1,), (1,)), ((), ())),
                         preferred_element_type=jnp.float32) + bih_ref[...]
    gh = lax.dot_general(iw, whh_ref[...], (((1,), (1,)), ((), ())),
                         preferred_element_type=jnp.float32) + bhh_ref[...]
    i_r, i_z, i_n = gi[:, :C], gi[:, C:2 * C], gi[:, 2 * C:]
    h_r, h_z, h_n = gh[:, :C], gh[:, C:2 * C], gh[:, 2 * C:]
    r = jax.nn.sigmoid(i_r + h_r)
    z = jax.nn.sigmoid(i_z + h_z)
    n = jnp.tanh(i_n + r * h_n)
    W = (1.0 - z) * n + z * iw

    deg = 1.0 + jnp.sum(dp_ref[...], axis=0)
    dinv = lax.rsqrt(deg)
    dinv2 = dinv[:, None]
    h2_ref[...] = dinv2 * jnp.dot(x_ref[...], W,
                                  preferred_element_type=jnp.float32)
    dinv_ref[...] = dinv2


@jax.jit
def _dense1_call(x, iw, wih, whh, bih, bhh, dp):
    return pl.pallas_call(
        _dense1_body,
        out_shape=(
            jax.ShapeDtypeStruct((N, C), jnp.float32),
            jax.ShapeDtypeStruct((N, 1), jnp.float32),
        ),
    )(x, iw, wih, whh, bih, bhh, dp)


NB = 2
MAIN = (NCHUNK - 1) // NB
TAIL = NCHUNK - 1
assert NB * MAIN == NCHUNK - 1


def _scale_chunk(buf, ew_v, g):
    def grp(q, carry):
        wv = ew_v[g, pl.ds(q * L, L)]
        for r2 in range(L):
            w = wv[r2]
            row = q * L + r2
            for jj in range(C // L):
                sl = pl.ds(jj * L, L)
                buf[row, sl] = buf[row, sl] * w
        return carry

    lax.fori_loop(0, K // L, grp, 0)


def _edge_body(row_hbm, col_hbm, ew_hbm, h2_hbm, zeros_hbm, out_hbm,
               row_v, col_v, ew_v, bufs, sems_g, sems_s, acc_sh):
    c = lax.axis_index("c")
    s = lax.axis_index("s")
    wid = c * NS + s

    pltpu.sync_copy(zeros_hbm.at[pl.ds(s * RPT, RPT)],
                    acc_sh.at[pl.ds(s * RPT, RPT)])
    pltpu.sync_copy(row_hbm.at[wid], row_v)
    pltpu.sync_copy(col_hbm.at[wid], col_v)
    pltpu.sync_copy(ew_hbm.at[wid], ew_v)
    plsc.subcore_barrier()

    def gather(g, j):
        pltpu.async_copy(h2_hbm.at[row_v.at[g]], bufs[j], sems_g[j])

    def scatter(g, j):
        pltpu.async_copy(bufs[j], acc_sh.at[col_v.at[g]], sems_s[j], add=True)

    def wait_gather(j):
        pltpu.make_async_copy(h2_hbm.at[row_v.at[0]], bufs[j],
                              sems_g[j]).wait()

    def wait_scatter(j):
        pltpu.make_async_copy(bufs[j], acc_sh.at[col_v.at[0]],
                              sems_s[j]).wait()

    for j in range(NB - 1):
        gather(j, j)

    def ring_body(i, carry):
        for j in range(NB):
            g = i * NB + j
            jp = (j + NB - 1) % NB
            if j == 0:
                @pl.when(i > 0)
                def _():
                    wait_scatter(jp)

                gather(g + NB - 1, jp)
            else:
                @pl.when(i < MAIN - 1)
                def _():
                    wait_scatter(jp)
                    gather(g + NB - 1, jp)

            wait_gather(j)
            _scale_chunk(bufs[j], ew_v, g)
            scatter(g, j)
        return carry

    lax.fori_loop(0, MAIN, ring_body, 0)

    wait_scatter(0)
    pltpu.sync_copy(h2_hbm.at[row_v.at[TAIL]], bufs[0])
    _scale_chunk(bufs[0], ew_v, TAIL)
    for j in range(1, NB):
        wait_scatter(j)
    pltpu.sync_copy(bufs[0], acc_sh.at[col_v.at[TAIL]], add=True)

    plsc.subcore_barrier()
    pltpu.sync_copy(acc_sh.at[pl.ds(s * RPT, RPT)],
                    out_hbm.at[c, pl.ds(s * RPT, RPT)])


@jax.jit
def _edge_call(row3, col3, ew3, h2, zeros):
    return pl.kernel(
        _edge_body,
        out_type=jax.ShapeDtypeStruct((NC, N, C), jnp.float32),
        mesh=_mesh(),
        compiler_params=_SC_PARAMS,
        scratch_types=[
            pltpu.VMEM((NCHUNK, K), jnp.int32),
            pltpu.VMEM((NCHUNK, K), jnp.int32),
            pltpu.VMEM((NCHUNK, K), jnp.float32),
            [pltpu.VMEM((K, C), jnp.float32)] * NB,
            [pltpu.SemaphoreType.DMA] * NB,
            [pltpu.SemaphoreType.DMA] * NB,
            pltpu.VMEM_SHARED((N, C), jnp.float32),
        ],
    )(row3, col3, ew3, h2, zeros)


def _dense2_body(acc_ref, h2_ref, dinv_ref, linw_ref, linb_ref, out_ref):
    pre = dinv_ref[...] * (acc_ref[0] + acc_ref[1] + h2_ref[...])
    pre = jnp.maximum(pre, 0.0)
    out_ref[...] = lax.dot_general(pre, linw_ref[...], (((1,), (1,)), ((), ())),
                                   preferred_element_type=jnp.float32
                                   ) + linb_ref[...]


@jax.jit
def _dense2_call(acc, h2, dinv2, lin_W, lin_b):
    return pl.pallas_call(
        _dense2_body,
        out_shape=jax.ShapeDtypeStruct((N, C), jnp.float32),
    )(acc, h2, dinv2, lin_W, lin_b)


def kernel(x, edge_index, edge_weight, init_weight, W_ih, W_hh, b_ih, b_hh,
           lin_W, lin_b):
    ei = edge_index.astype(jnp.int32)
    row = ei[0]
    col = ei[1]
    dp = _deg_call(col, edge_weight)
    h2, dinv2 = _dense1_call(x, init_weight, W_ih, W_hh, b_ih, b_hh, dp)
    zeros = jnp.zeros((N, C), jnp.float32)
    pad = E_PAD - E
    pad_idx = (jnp.arange(pad, dtype=jnp.int32) * 16) % N
    row3 = jnp.concatenate([row, pad_idx]).reshape(NW, NCHUNK, K)
    col3 = jnp.concatenate([col, pad_idx]).reshape(NW, NCHUNK, K)
    ew3 = jnp.concatenate(
        [edge_weight, jnp.zeros((pad,), jnp.float32)]).reshape(NW, NCHUNK, K)
    acc = _edge_call(row3, col3, ew3, h2, zeros)
    return _dense2_call(acc, h2, dinv2, lin_W, lin_b)


# speedup vs baseline: 1.1469x; 1.1469x over previous
import functools

import jax
import jax.numpy as jnp
from jax import lax
from jax.experimental import pallas as pl
from jax.experimental.pallas import tpu as pltpu
from jax.experimental.pallas import tpu_sc as plsc

N = 10000
C = 128
E = 320000

NC = 2
NS = 16
NW = NC * NS
EW = E // NW
K = 80
KE = 96
EWP = 10160
E_PAD = EWP * NW
NCHUNK = EWP // K
RPT = N // NS
L = 16

_mesh = functools.partial(
    plsc.VectorSubcoreMesh, core_axis_name="c", subcore_axis_name="s")
_SC_PARAMS = pltpu.CompilerParams(needs_layout_passes=False,
                                  use_tc_tiling_on_sc=False)


def _deg_body(col_hbm, ew_hbm, out_hbm, col_v, ew_v, deg_v):
    c = lax.axis_index("c")
    s = lax.axis_index("s")
    wid = c * NS + s
    base = wid * EW
    pltpu.sync_copy(col_hbm.at[pl.ds(base, EW)], col_v)
    pltpu.sync_copy(ew_hbm.at[pl.ds(base, EW)], ew_v)

    def zero_body(i, carry):
        deg_v[pl.ds(i * L, L)] = jnp.zeros((L,), jnp.float32)
        return carry

    lax.fori_loop(0, N // L, zero_body, 0)

    def acc_body(i, carry):
        idx = col_v[pl.ds(i * L, L)]
        w = ew_v[pl.ds(i * L, L)]
        plsc.addupdate_scatter(deg_v, [idx], w)
        return carry

    lax.fori_loop(0, EW // L, acc_body, 0)
    pltpu.sync_copy(deg_v, out_hbm.at[wid])


@jax.jit
def _deg_call(col, ew):
    return pl.kernel(
        _deg_body,
        out_type=jax.ShapeDtypeStruct((NW, N), jnp.float32),
        mesh=_mesh(),
        compiler_params=_SC_PARAMS,
        scratch_types=[
            pltpu.VMEM((EW,), jnp.int32),
            pltpu.VMEM((EW,), jnp.float32),
            pltpu.VMEM((N,), jnp.float32),
        ],
    )(col, ew)


def _dense1_body(x_ref, iw_ref, wih_ref, whh_ref, bih_ref, bhh_ref, dp_ref,
                 h2_ref, dinv_ref):
    iw = iw_ref[...]
    gi = lax.dot_general(iw, wih_ref[...], (((1,), (1,)), ((), ())),
                         preferred_element_type=jnp.float32) + bih_ref[...]
    gh = lax.dot_general(iw, whh_ref[...], (((1,), (1,)), ((), ())),
                         preferred_element_type=jnp.float32) + bhh_ref[...]
    i_r, i_z, i_n = gi[:, :C], gi[:, C:2 * C], gi[:, 2 * C:]
    h_r, h_z, h_n = gh[:, :C], gh[:, C:2 * C], gh[:, 2 * C:]
    r = jax.nn.sigmoid(i_r + h_r)
    z = jax.nn.sigmoid(i_z + h_z)
    n = jnp.tanh(i_n + r * h_n)
    W = (1.0 - z) * n + z * iw

    deg = 1.0 + jnp.sum(dp_ref[...], axis=0)
    dinv = lax.rsqrt(deg)
    dinv2 = dinv[:, None]
    h2_ref[...] = dinv2 * jnp.dot(x_ref[...], W,
                                  preferred_element_type=jnp.float32)
    dinv_ref[...] = dinv2


@jax.jit
def _dense1_call(x, iw, wih, whh, bih, bhh, dp):
    return pl.pallas_call(
        _dense1_body,
        out_shape=(
            jax.ShapeDtypeStruct((N, C), jnp.float32),
            jax.ShapeDtypeStruct((N, 1), jnp.float32),
        ),
    )(x, iw, wih, whh, bih, bhh, dp)


NB = 3
MAIN = (NCHUNK - 1) // NB
TAIL = NCHUNK - 1
assert NB * MAIN == NCHUNK - 1


def _rows_scaled(buf, wv, r0):
    for r2 in range(L):
        w = wv[r2]
        row = r0 + r2
        for jj in range(C // L):
            sl = pl.ds(jj * L, L)
            buf[row, sl] = buf[row, sl] * w


def _scale_chunk(buf, ew_v, g):
    def grp32(q, carry):
        ewv = ew_v[g, pl.ds(q * 2 * L, 2 * L)]
        a, b = plsc.unpack(ewv, format=plsc.PackFormat.INTERLEAVED)
        _rows_scaled(buf, a, q * 2 * L)
        _rows_scaled(buf, b, q * 2 * L + L)
        return carry

    lax.fori_loop(0, K // (2 * L), grp32, 0)
    if K % (2 * L):
        ewv = ew_v[g, pl.ds(K - L, 2 * L)]
        a, _ = plsc.unpack(ewv, format=plsc.PackFormat.INTERLEAVED)
        _rows_scaled(buf, a, K - L)


def _edge_body(packed_hbm, ew_hbm, h2_hbm, zeros_hbm, out_hbm,
               packed_v, ew_v, bufs, rowr, colr, sems_g, sems_s, acc_sh):
    c = lax.axis_index("c")
    s = lax.axis_index("s")
    wid = c * NS + s

    pltpu.sync_copy(zeros_hbm.at[pl.ds(s * RPT, RPT)],
                    acc_sh.at[pl.ds(s * RPT, RPT)])
    pltpu.sync_copy(packed_hbm.at[wid], packed_v)
    pltpu.sync_copy(ew_hbm.at[wid], ew_v)
    plsc.subcore_barrier()

    def gather(g, j):
        for q in range(K // L):
            sl = pl.ds(q * L, L)
            rowr[j][sl] = packed_v[g, sl] & 0x3FFF
        pltpu.async_copy(h2_hbm.at[rowr[j]], bufs[j], sems_g[j])

    def scatter(g, j):
        for q in range(K // L):
            sl = pl.ds(q * L, L)
            colr[j][sl] = lax.shift_right_logical(packed_v[g, sl], 14)
        pltpu.async_copy(bufs[j], acc_sh.at[colr[j]], sems_s[j], add=True)

    def wait_gather(j):
        pltpu.make_async_copy(h2_hbm.at[rowr[j]], bufs[j], sems_g[j]).wait()

    def wait_scatter(j):
        pltpu.make_async_copy(bufs[j], acc_sh.at[colr[j]], sems_s[j]).wait()

    for j in range(NB - 1):
        gather(j, j)

    def ring_body(i, carry):
        for j in range(NB):
            g = i * NB + j
            jp = (j + NB - 1) % NB
            if j == 0:
                @pl.when(i > 0)
                def _():
                    wait_scatter(jp)

                gather(g + NB - 1, jp)
            else:
                @pl.when(i < MAIN - 1)
                def _():
                    wait_scatter(jp)
                    gather(g + NB - 1, jp)

            wait_gather(j)
            _scale_chunk(bufs[j], ew_v, g)
            scatter(g, j)
        return carry

    lax.fori_loop(0, MAIN, ring_body, 0)

    wait_scatter(0)
    gather(TAIL, 0)
    wait_gather(0)
    _scale_chunk(bufs[0], ew_v, TAIL)
    for j in range(1, NB):
        wait_scatter(j)
    scatter(TAIL, 0)
    wait_scatter(0)

    plsc.subcore_barrier()
    pltpu.sync_copy(acc_sh.at[pl.ds(s * RPT, RPT)],
                    out_hbm.at[c, pl.ds(s * RPT, RPT)])


@jax.jit
def _edge_call(packed3, ew3, h2, zeros):
    return pl.kernel(
        _edge_body,
        out_type=jax.ShapeDtypeStruct((NC, N, C), jnp.float32),
        mesh=_mesh(),
        compiler_params=_SC_PARAMS,
        scratch_types=[
            pltpu.VMEM((NCHUNK, K), jnp.int32),
            pltpu.VMEM((NCHUNK, KE), jnp.bfloat16),
            [pltpu.VMEM((K, C), jnp.float32)] * NB,
            [pltpu.VMEM((K,), jnp.int32)] * NB,
            [pltpu.VMEM((K,), jnp.int32)] * NB,
            [pltpu.SemaphoreType.DMA] * NB,
            [pltpu.SemaphoreType.DMA] * NB,
            pltpu.VMEM_SHARED((N, C), jnp.float32),
        ],
    )(packed3, ew3, h2, zeros)


def _dense2_body(acc_ref, h2_ref, dinv_ref, linw_ref, linb_ref, out_ref):
    pre = dinv_ref[...] * (acc_ref[0] + acc_ref[1] + h2_ref[...])
    pre = jnp.maximum(pre, 0.0)
    out_ref[...] = lax.dot_general(pre, linw_ref[...], (((1,), (1,)), ((), ())),
                                   preferred_element_type=jnp.float32
                                   ) + linb_ref[...]


@jax.jit
def _dense2_call(acc, h2, dinv2, lin_W, lin_b):
    return pl.pallas_call(
        _dense2_body,
        out_shape=jax.ShapeDtypeStruct((N, C), jnp.float32),
    )(acc, h2, dinv2, lin_W, lin_b)


def kernel(x, edge_index, edge_weight, init_weight, W_ih, W_hh, b_ih, b_hh,
           lin_W, lin_b):
    ei = edge_index.astype(jnp.int32)
    row = ei[0]
    col = ei[1]
    dp = _deg_call(col, edge_weight)
    h2, dinv2 = _dense1_call(x, init_weight, W_ih, W_hh, b_ih, b_hh, dp)
    zeros = jnp.zeros((N, C), jnp.float32)
    pad = E_PAD - E
    pad_idx = (jnp.arange(pad, dtype=jnp.int32) * 16) % N
    row_p = jnp.concatenate([row, pad_idx])
    col_p = jnp.concatenate([col, pad_idx])
    packed3 = (row_p | (col_p << 14)).reshape(NW, NCHUNK, K)
    ew_p = jnp.concatenate(
        [edge_weight, jnp.zeros((pad,), jnp.float32)]
    ).astype(jnp.bfloat16).reshape(NW, NCHUNK, K)
    ew_pk = jnp.concatenate(
        [ew_p, jnp.zeros((NW, NCHUNK, KE - K), jnp.bfloat16)], axis=-1)
    ew3 = ew_pk.reshape(NW, NCHUNK, KE // 32, 2, L).transpose(
        0, 1, 2, 4, 3).reshape(NW, NCHUNK, KE)
    acc = _edge_call(packed3, ew3, h2, zeros)
    return _dense2_call(acc, h2, dinv2, lin_W, lin_b)
